# baseline (device time: 110154 ns/iter reference)
import jax
import jax.numpy as jnp
from jax import lax
from jax.experimental import pallas as pl
from jax.experimental.pallas import tpu as pltpu

N_DEV = 8
N_TOK = 2048
D_MODEL = 512
D_FF = 1024
N_EXP = 32
E_LOCAL = 4
CHUNK = N_TOK // N_DEV


def kernel(x, router_W, route_idx, expert_W):
    def body(x_ref, rw_ref, idx_ref, ew_ref, out_ref,
             comm_ref, gates_ref, ew_bf_ref, send_sems, recv_sems):
        my = lax.axis_index("i")
        left = lax.rem(my - 1 + N_DEV, N_DEV)
        right = lax.rem(my + 1, N_DEV)

        scores = jnp.dot(x_ref[:, :], rw_ref[:, :],
                         preferred_element_type=jnp.float32)
        m = jnp.max(scores, axis=1, keepdims=True)
        p = jnp.exp(scores - m)
        probs = p / jnp.sum(p, axis=1, keepdims=True)
        e_iota = lax.broadcasted_iota(jnp.int32, (N_TOK, N_EXP), 1)
        idx0 = idx_ref[:, 0:1]
        idx1 = idx_ref[:, 1:2]
        g0 = jnp.sum(jnp.where(e_iota == idx0, probs, 0.0), axis=1,
                     keepdims=True)
        g1 = jnp.sum(jnp.where(e_iota == idx1, probs, 0.0), axis=1,
                     keepdims=True)
        gs = g0 + g1
        gates_ref[:, 0:1] = g0 / gs
        gates_ref[:, 1:2] = g1 / gs

        ew_bf_ref[...] = ew_ref[...].astype(jnp.bfloat16)

        def chunk_partial(c):
            rows = pl.ds(c * CHUNK, CHUNK)
            xr = x_ref[rows, :]
            xi0 = idx_ref[rows, 0:1]
            xi1 = idx_ref[rows, 1:2]
            w0 = gates_ref[rows, 0:1]
            w1 = gates_ref[rows, 1:2]
            acc = jnp.zeros((CHUNK, D_FF), jnp.float32)
            for j in range(E_LOCAL):
                ge = my * E_LOCAL + j
                coeff = (jnp.where(xi0 == ge, w0, 0.0)
                         + jnp.where(xi1 == ge, w1, 0.0))
                acc = acc + jnp.dot((xr * coeff).astype(jnp.bfloat16),
                                    ew_bf_ref[j],
                                    preferred_element_type=jnp.float32)
            return acc

        barrier = pltpu.get_barrier_semaphore()
        for nbr in (left, right):
            pl.semaphore_signal(barrier, inc=1, device_id=(nbr,),
                                device_id_type=pl.DeviceIdType.MESH)
        pl.semaphore_wait(barrier, 2)

        comm_ref[0, :, :] = chunk_partial(left)

        for s in range(N_DEV - 1):
            rdma = pltpu.make_async_remote_copy(
                src_ref=comm_ref.at[s],
                dst_ref=comm_ref.at[s + 1],
                send_sem=send_sems.at[s],
                recv_sem=recv_sems.at[s],
                device_id=(right,),
                device_id_type=pl.DeviceIdType.MESH,
            )
            rdma.start()
            c = lax.rem(my - 2 - s + 2 * N_DEV, N_DEV)
            part = chunk_partial(c)
            rdma.wait()
            if s < N_DEV - 2:
                comm_ref[s + 1, :, :] = comm_ref[s + 1, :, :] + part
            else:
                out_ref[:, :] = comm_ref[s + 1, :, :] + part

    return pl.pallas_call(
        body,
        out_shape=jax.ShapeDtypeStruct((CHUNK, D_FF), jnp.float32),
        in_specs=[
            pl.BlockSpec(memory_space=pltpu.VMEM),
            pl.BlockSpec(memory_space=pltpu.VMEM),
            pl.BlockSpec(memory_space=pltpu.VMEM),
            pl.BlockSpec(memory_space=pltpu.VMEM),
        ],
        out_specs=pl.BlockSpec(memory_space=pltpu.VMEM),
        scratch_shapes=[
            pltpu.VMEM((N_DEV, CHUNK, D_FF), jnp.float32),
            pltpu.VMEM((N_TOK, 2), jnp.float32),
            pltpu.VMEM((E_LOCAL, D_MODEL, D_FF), jnp.bfloat16),
            pltpu.SemaphoreType.DMA((N_DEV - 1,)),
            pltpu.SemaphoreType.DMA((N_DEV - 1,)),
        ],
        compiler_params=pltpu.CompilerParams(collective_id=0),
    )(x, router_W, route_idx, expert_W)


# device time: 73005 ns/iter; 1.5089x vs baseline; 1.5089x over previous
import jax
import jax.numpy as jnp
from jax import lax
from jax.experimental import pallas as pl
from jax.experimental.pallas import tpu as pltpu

N_DEV = 8
N_TOK = 2048
D_MODEL = 512
D_FF = 1024
HALF = D_FF // 2
N_EXP = 32
E_LOCAL = 4
CHUNK = N_TOK // N_DEV


def kernel(x, router_W, route_idx, expert_W):
    def body(x_ref, rw_ref, idx_ref, ew_ref, out_ref,
             cw_ref, ccw_ref, gates_ref, ew_bf_ref,
             send_cw, recv_cw, send_ccw, recv_ccw):
        my = lax.axis_index("i")
        left = lax.rem(my - 1 + N_DEV, N_DEV)
        right = lax.rem(my + 1, N_DEV)

        scores = jnp.dot(x_ref[:, :], rw_ref[:, :],
                         preferred_element_type=jnp.float32)
        m = jnp.max(scores, axis=1, keepdims=True)
        p = jnp.exp(scores - m)
        probs = p / jnp.sum(p, axis=1, keepdims=True)
        e_iota = lax.broadcasted_iota(jnp.int32, (N_TOK, N_EXP), 1)
        idx0 = idx_ref[:, 0:1]
        idx1 = idx_ref[:, 1:2]
        g0 = jnp.sum(jnp.where(e_iota == idx0, probs, 0.0), axis=1,
                     keepdims=True)
        g1 = jnp.sum(jnp.where(e_iota == idx1, probs, 0.0), axis=1,
                     keepdims=True)
        gs = g0 + g1
        gates_ref[:, 0:1] = g0 / gs
        gates_ref[:, 1:2] = g1 / gs

        ew_bf_ref[...] = ew_ref[...].astype(jnp.bfloat16)

        def chunk_partial(c, h):
            rows = pl.ds(c * CHUNK, CHUNK)
            cols = slice(h * HALF, (h + 1) * HALF)
            xr = x_ref[rows, :]
            xi0 = idx_ref[rows, 0:1]
            xi1 = idx_ref[rows, 1:2]
            w0 = gates_ref[rows, 0:1]
            w1 = gates_ref[rows, 1:2]
            acc = jnp.zeros((CHUNK, HALF), jnp.float32)
            for j in range(E_LOCAL):
                ge = my * E_LOCAL + j
                coeff = (jnp.where(xi0 == ge, w0, 0.0)
                         + jnp.where(xi1 == ge, w1, 0.0))
                acc = acc + jnp.dot((xr * coeff).astype(jnp.bfloat16),
                                    ew_bf_ref[j, :, cols],
                                    preferred_element_type=jnp.float32)
            return acc

        barrier = pltpu.get_barrier_semaphore()
        for nbr in (left, right):
            pl.semaphore_signal(barrier, inc=1, device_id=(nbr,),
                                device_id_type=pl.DeviceIdType.MESH)
        pl.semaphore_wait(barrier, 2)

        cw_ref[0, :, :] = chunk_partial(left, 0)
        ccw_ref[0, :, :] = chunk_partial(right, 1)

        for s in range(N_DEV - 1):
            rdma_cw = pltpu.make_async_remote_copy(
                src_ref=cw_ref.at[s],
                dst_ref=cw_ref.at[s + 1],
                send_sem=send_cw.at[s],
                recv_sem=recv_cw.at[s],
                device_id=(right,),
                device_id_type=pl.DeviceIdType.MESH,
            )
            rdma_ccw = pltpu.make_async_remote_copy(
                src_ref=ccw_ref.at[s],
                dst_ref=ccw_ref.at[s + 1],
                send_sem=send_ccw.at[s],
                recv_sem=recv_ccw.at[s],
                device_id=(left,),
                device_id_type=pl.DeviceIdType.MESH,
            )
            rdma_cw.start()
            rdma_ccw.start()
            c_cw = lax.rem(my - 2 - s + 2 * N_DEV, N_DEV)
            c_ccw = lax.rem(my + 2 + s, N_DEV)
            part_cw = chunk_partial(c_cw, 0)
            part_ccw = chunk_partial(c_ccw, 1)
            rdma_cw.wait()
            rdma_ccw.wait()
            if s < N_DEV - 2:
                cw_ref[s + 1, :, :] = cw_ref[s + 1, :, :] + part_cw
                ccw_ref[s + 1, :, :] = ccw_ref[s + 1, :, :] + part_ccw
            else:
                out_ref[:, 0:HALF] = cw_ref[s + 1, :, :] + part_cw
                out_ref[:, HALF:D_FF] = ccw_ref[s + 1, :, :] + part_ccw

    return pl.pallas_call(
        body,
        out_shape=jax.ShapeDtypeStruct((CHUNK, D_FF), jnp.float32),
        in_specs=[
            pl.BlockSpec(memory_space=pltpu.VMEM),
            pl.BlockSpec(memory_space=pltpu.VMEM),
            pl.BlockSpec(memory_space=pltpu.VMEM),
            pl.BlockSpec(memory_space=pltpu.VMEM),
        ],
        out_specs=pl.BlockSpec(memory_space=pltpu.VMEM),
        scratch_shapes=[
            pltpu.VMEM((N_DEV, CHUNK, HALF), jnp.float32),
            pltpu.VMEM((N_DEV, CHUNK, HALF), jnp.float32),
            pltpu.VMEM((N_TOK, 2), jnp.float32),
            pltpu.VMEM((E_LOCAL, D_MODEL, D_FF), jnp.bfloat16),
            pltpu.SemaphoreType.DMA((N_DEV - 1,)),
            pltpu.SemaphoreType.DMA((N_DEV - 1,)),
            pltpu.SemaphoreType.DMA((N_DEV - 1,)),
            pltpu.SemaphoreType.DMA((N_DEV - 1,)),
        ],
        compiler_params=pltpu.CompilerParams(collective_id=0),
    )(x, router_W, route_idx, expert_W)


# device time: 53650 ns/iter; 2.0532x vs baseline; 1.3608x over previous
import jax
import jax.numpy as jnp
from jax import lax
from jax.experimental import pallas as pl
from jax.experimental.pallas import tpu as pltpu

N_DEV = 8
N_TOK = 2048
D_MODEL = 512
D_FF = 1024
HALF = D_FF // 2
N_EXP = 32
E_LOCAL = 4
CHUNK = N_TOK // N_DEV


def kernel(x, router_W, route_idx, expert_W):
    def body(x_ref, rw_ref, idx_ref, ew_ref, out_ref,
             cw_ref, ccw_ref, gates_ref, ew_bf_ref,
             send_cw, recv_cw, send_ccw, recv_ccw):
        my = lax.axis_index("i")
        left = lax.rem(my - 1 + N_DEV, N_DEV)
        right = lax.rem(my + 1, N_DEV)

        scores = jnp.dot(x_ref[:, :], rw_ref[:, :],
                         preferred_element_type=jnp.float32)
        m = jnp.max(scores, axis=1, keepdims=True)
        p = jnp.exp(scores - m)
        probs = p / jnp.sum(p, axis=1, keepdims=True)
        e_iota = lax.broadcasted_iota(jnp.int32, (N_TOK, N_EXP), 1)
        idx0 = idx_ref[:, 0:1]
        idx1 = idx_ref[:, 1:2]
        g0 = jnp.sum(jnp.where(e_iota == idx0, probs, 0.0), axis=1,
                     keepdims=True)
        g1 = jnp.sum(jnp.where(e_iota == idx1, probs, 0.0), axis=1,
                     keepdims=True)
        gs = g0 + g1
        gates_ref[:, 0:1] = g0 / gs
        gates_ref[:, 1:2] = g1 / gs

        ew_bf_ref[...] = ew_ref[...].astype(jnp.bfloat16)

        def chunk_partial(c, h):
            rows = pl.ds(c * CHUNK, CHUNK)
            cols = slice(h * HALF, (h + 1) * HALF)
            xr = x_ref[rows, :]
            xi0 = idx_ref[rows, 0:1]
            xi1 = idx_ref[rows, 1:2]
            w0 = gates_ref[rows, 0:1]
            w1 = gates_ref[rows, 1:2]
            acc = jnp.zeros((CHUNK, HALF), jnp.float32)
            for j in range(E_LOCAL):
                ge = my * E_LOCAL + j
                coeff = (jnp.where(xi0 == ge, w0, 0.0)
                         + jnp.where(xi1 == ge, w1, 0.0))
                acc = acc + jnp.dot((xr * coeff).astype(jnp.bfloat16),
                                    ew_bf_ref[j, :, cols],
                                    preferred_element_type=jnp.float32)
            return acc

        barrier = pltpu.get_barrier_semaphore()
        for nbr in (left, right):
            pl.semaphore_signal(barrier, inc=1, device_id=(nbr,),
                                device_id_type=pl.DeviceIdType.MESH)
        pl.semaphore_wait(barrier, 2)

        cw_ref[0, :, :] = chunk_partial(left, 0).astype(jnp.bfloat16)
        ccw_ref[0, :, :] = chunk_partial(right, 1).astype(jnp.bfloat16)

        for s in range(N_DEV - 1):
            rdma_cw = pltpu.make_async_remote_copy(
                src_ref=cw_ref.at[s],
                dst_ref=cw_ref.at[s + 1],
                send_sem=send_cw.at[s],
                recv_sem=recv_cw.at[s],
                device_id=(right,),
                device_id_type=pl.DeviceIdType.MESH,
            )
            rdma_ccw = pltpu.make_async_remote_copy(
                src_ref=ccw_ref.at[s],
                dst_ref=ccw_ref.at[s + 1],
                send_sem=send_ccw.at[s],
                recv_sem=recv_ccw.at[s],
                device_id=(left,),
                device_id_type=pl.DeviceIdType.MESH,
            )
            rdma_cw.start()
            rdma_ccw.start()
            c_cw = lax.rem(my - 2 - s + 2 * N_DEV, N_DEV)
            c_ccw = lax.rem(my + 2 + s, N_DEV)
            part_cw = chunk_partial(c_cw, 0)
            part_ccw = chunk_partial(c_ccw, 1)
            rdma_cw.wait()
            rdma_ccw.wait()
            if s < N_DEV - 2:
                cw_ref[s + 1, :, :] = (
                    cw_ref[s + 1, :, :].astype(jnp.float32) + part_cw
                ).astype(jnp.bfloat16)
                ccw_ref[s + 1, :, :] = (
                    ccw_ref[s + 1, :, :].astype(jnp.float32) + part_ccw
                ).astype(jnp.bfloat16)
            else:
                out_ref[:, 0:HALF] = (
                    cw_ref[s + 1, :, :].astype(jnp.float32) + part_cw
                )
                out_ref[:, HALF:D_FF] = (
                    ccw_ref[s + 1, :, :].astype(jnp.float32) + part_ccw
                )

    return pl.pallas_call(
        body,
        out_shape=jax.ShapeDtypeStruct((CHUNK, D_FF), jnp.float32),
        in_specs=[
            pl.BlockSpec(memory_space=pltpu.VMEM),
            pl.BlockSpec(memory_space=pltpu.VMEM),
            pl.BlockSpec(memory_space=pltpu.VMEM),
            pl.BlockSpec(memory_space=pltpu.VMEM),
        ],
        out_specs=pl.BlockSpec(memory_space=pltpu.VMEM),
        scratch_shapes=[
            pltpu.VMEM((N_DEV, CHUNK, HALF), jnp.bfloat16),
            pltpu.VMEM((N_DEV, CHUNK, HALF), jnp.bfloat16),
            pltpu.VMEM((N_TOK, 2), jnp.float32),
            pltpu.VMEM((E_LOCAL, D_MODEL, D_FF), jnp.bfloat16),
            pltpu.SemaphoreType.DMA((N_DEV - 1,)),
            pltpu.SemaphoreType.DMA((N_DEV - 1,)),
            pltpu.SemaphoreType.DMA((N_DEV - 1,)),
            pltpu.SemaphoreType.DMA((N_DEV - 1,)),
        ],
        compiler_params=pltpu.CompilerParams(collective_id=0),
    )(x, router_W, route_idx, expert_W)
